# Initial kernel scaffold; baseline (speedup 1.0000x reference)
import functools

import jax
import jax.numpy as jnp
from jax import lax
from jax.experimental import pallas as pl
from jax.experimental.pallas import tpu as pltpu
from jax.experimental.pallas import tpu_sc as plsc

N = 10000
E = 320000
D = 128
NC = 2
NS = 16
NW = NC * NS
CH = 128
NCHUNK = -(-E // (NW * CH))
EPAD = NW * NCHUNK * CH
ROWS = 10016
RPT = ROWS // NS
DEGC = 16

_mesh = plsc.VectorSubcoreMesh(core_axis_name="c", subcore_axis_name="s")


def _deg_body(dst3_hbm, out_hbm, dstv, onesv, zbuf, deg_sh, sem):
    del sem
    c = lax.axis_index("c")
    s = lax.axis_index("s")
    wid = c * NS + s
    pltpu.sync_copy(dst3_hbm.at[wid], dstv)

    def fill(i, carry):
        onesv[i] = jnp.full((16,), 1.0, jnp.float32)
        return carry

    lax.fori_loop(0, CH, fill, 0)

    def zero(i, carry):
        zbuf[i] = jnp.zeros((16,), jnp.float32)
        return carry

    lax.fori_loop(0, RPT, zero, 0)
    pltpu.sync_copy(zbuf, deg_sh.at[pl.ds(s * RPT, RPT)])
    plsc.subcore_barrier()

    def scat(j, carry):
        pltpu.sync_copy(onesv, deg_sh.at[dstv.at[j]], add=True)
        return carry

    lax.fori_loop(0, NCHUNK, scat, 0)
    plsc.subcore_barrier()
    pltpu.sync_copy(deg_sh.at[pl.ds(s * RPT, RPT)],
                    out_hbm.at[c, pl.ds(s * RPT, RPT)])


@functools.partial(
    pl.kernel,
    out_type=jax.ShapeDtypeStruct((NC, ROWS, DEGC), jnp.float32),
    mesh=_mesh,
    scratch_types=[
        pltpu.VMEM((NCHUNK, CH), jnp.int32),
        pltpu.VMEM((CH, DEGC), jnp.float32),
        pltpu.VMEM((RPT, DEGC), jnp.float32),
        pltpu.VMEM_SHARED((ROWS, DEGC), jnp.float32),
        pltpu.SemaphoreType.DMA,
    ],
    name="sc_degree",
)
def _deg_kernel(dst3_hbm, out_hbm, dstv, onesv, zbuf, deg_sh, sem):
    _deg_body(dst3_hbm, out_hbm, dstv, onesv, zbuf, deg_sh, sem)


def _scat_body(h2_hbm, src3_hbm, dst3_hbm, out_hbm,
               srcv, dstv, rows, zbuf, acc_sh, sem):
    c = lax.axis_index("c")
    s = lax.axis_index("s")
    wid = c * NS + s
    pltpu.sync_copy(src3_hbm.at[wid], srcv)
    pltpu.sync_copy(dst3_hbm.at[wid], dstv)

    def zero(i, carry):
        for k in range(D // 16):
            zbuf[i, pl.ds(k * 16, 16)] = jnp.zeros((16,), jnp.float32)
        return carry

    lax.fori_loop(0, RPT // 2, zero, 0)
    pltpu.sync_copy(zbuf, acc_sh.at[pl.ds(s * RPT, RPT // 2)])
    pltpu.sync_copy(zbuf, acc_sh.at[pl.ds(s * RPT + RPT // 2, RPT // 2)])
    plsc.subcore_barrier()

    def go(j, carry):
        pltpu.async_copy(h2_hbm.at[srcv.at[j]], rows, sem).wait()
        pltpu.sync_copy(rows, acc_sh.at[dstv.at[j]], add=True)
        return carry

    lax.fori_loop(0, NCHUNK, go, 0)
    plsc.subcore_barrier()
    pltpu.sync_copy(acc_sh.at[pl.ds(s * RPT, RPT)],
                    out_hbm.at[c, pl.ds(s * RPT, RPT)])


@functools.partial(
    pl.kernel,
    out_type=jax.ShapeDtypeStruct((NC, ROWS, D), jnp.float32),
    mesh=_mesh,
    scratch_types=[
        pltpu.VMEM((NCHUNK, CH), jnp.int32),
        pltpu.VMEM((NCHUNK, CH), jnp.int32),
        pltpu.VMEM((CH, D), jnp.float32),
        pltpu.VMEM((RPT // 2, D), jnp.float32),
        pltpu.VMEM_SHARED((ROWS, D), jnp.float32),
        pltpu.SemaphoreType.DMA,
    ],
    name="sc_edge_scatter",
)
def _scat_kernel(h2_hbm, src3_hbm, dst3_hbm, out_hbm,
                 srcv, dstv, rows, zbuf, acc_sh, sem):
    _scat_body(h2_hbm, src3_hbm, dst3_hbm, out_hbm,
               srcv, dstv, rows, zbuf, acc_sh, sem)


def _tc1_body(x_ref, w1_ref, degp_ref, h2_ref, dinv_ref):
    deg = (degp_ref[0, :N, :1] + degp_ref[1, :N, :1]) + 1.0
    dinv = lax.rsqrt(deg)
    h = jnp.dot(x_ref[...], w1_ref[...], preferred_element_type=jnp.float32)
    h2_ref[...] = h * dinv
    dinv_ref[...] = dinv


def _tc2_body(h2_ref, accp_ref, dinv_ref, w2_ref, b1_ref, h2b_ref):
    dinv = dinv_ref[...]
    acc = accp_ref[0, :N, :] + accp_ref[1, :N, :] + h2_ref[...]
    y1 = jnp.maximum(acc * dinv + b1_ref[...], 0.0)
    h = jnp.dot(y1, w2_ref[...], preferred_element_type=jnp.float32)
    h2b_ref[...] = h * dinv


def _tc3_body(h2b_ref, accp_ref, dinv_ref, b2_ref, lnw_ref, lnb_ref, out_ref):
    acc = accp_ref[0, :N, :] + accp_ref[1, :N, :] + h2b_ref[...]
    y = jnp.maximum(acc * dinv_ref[...] + b2_ref[...], 0.0)
    m = jnp.mean(y)
    o = y - m
    v = jnp.mean(o * o)
    out_ref[...] = (o / jnp.sqrt(v + 1e-5)) * lnw_ref[...] + lnb_ref[...]


_tc1 = pl.pallas_call(
    _tc1_body,
    out_shape=(jax.ShapeDtypeStruct((N, D), jnp.float32),
               jax.ShapeDtypeStruct((N, 1), jnp.float32)),
    name="tc_h2_dinv",
)

_tc2 = pl.pallas_call(
    _tc2_body,
    out_shape=jax.ShapeDtypeStruct((N, D), jnp.float32),
    name="tc_layer1",
)

_tc3 = pl.pallas_call(
    _tc3_body,
    out_shape=jax.ShapeDtypeStruct((N, D), jnp.float32),
    name="tc_layer2_ln",
)


def kernel(node_features, edges, W1, b1, W2, b2, ln_w, ln_b):
    src = edges[0]
    dst = edges[1]
    pad = EPAD - E
    srcp = jnp.concatenate([src, jnp.zeros((pad,), jnp.int32)])
    dstp = jnp.concatenate([dst, jnp.full((pad,), N, jnp.int32)])
    src3 = srcp.reshape(NW, NCHUNK, CH)
    dst3 = dstp.reshape(NW, NCHUNK, CH)

    degp = _deg_kernel(dst3)
    h2, dinv = _tc1(node_features, W1, degp)
    acc1 = _scat_kernel(h2, src3, dst3)
    h2b = _tc2(h2, acc1, dinv, W2, b1.reshape(1, D))
    acc2 = _scat_kernel(h2b, src3, dst3)
    out = _tc3(h2b, acc2, dinv, b2.reshape(1, D),
               ln_w.reshape(1, 1), ln_b.reshape(1, 1))
    return out


# speedup vs baseline: 7.4329x; 7.4329x over previous
import functools

import jax
import jax.numpy as jnp
from jax import lax
from jax.experimental import pallas as pl
from jax.experimental.pallas import tpu as pltpu
from jax.experimental.pallas import tpu_sc as plsc

N = 10000
E = 320000
D = 128
NC = 2
NS = 16
NW = NC * NS
CP = D // NW
BLK = 4096
NBLK = -(-E // BLK)
EPAD = NBLK * BLK
EPT = EPAD // NW
ROWS = 10016
SHIFT = 14
MASK = (1 << SHIFT) - 1

_mesh = plsc.VectorSubcoreMesh(core_axis_name="c", subcore_axis_name="s")
_sc_params = pltpu.CompilerParams(use_tc_tiling_on_sc=False,
                                  needs_layout_passes=False)


@functools.partial(
    pl.kernel,
    out_type=jax.ShapeDtypeStruct((NW, ROWS), jnp.float32),
    mesh=_mesh,
    scratch_types=[
        pltpu.VMEM((EPT,), jnp.int32),
        pltpu.VMEM((ROWS,), jnp.float32),
    ],
    compiler_params=_sc_params,
    name="sc_degree",
)
def _deg_kernel(pk_hbm, zeros_hbm, out_hbm, pkv, degv):
    c = lax.axis_index("c")
    s = lax.axis_index("s")
    wid = c * NS + s
    pltpu.sync_copy(pk_hbm.at[pl.ds(wid * EPT, EPT)], pkv)
    pltpu.sync_copy(zeros_hbm, degv)
    ones = jnp.ones((16,), jnp.float32)

    def go(g, carry):
        pk16 = pkv[pl.ds(g * 16, 16)]
        dst = jnp.bitwise_and(pk16, MASK)
        plsc.addupdate_scatter(degv, [dst], ones)
        return carry

    lax.fori_loop(0, EPT // 16, go, 0)
    pltpu.sync_copy(degv, out_hbm.at[wid])


@functools.partial(
    pl.kernel,
    out_type=jax.ShapeDtypeStruct((NW, CP, ROWS), jnp.float32),
    mesh=_mesh,
    scratch_types=[
        pltpu.VMEM((CP, N), jnp.float32),
        pltpu.VMEM((CP, ROWS), jnp.float32),
        pltpu.VMEM((BLK,), jnp.int32),
    ],
    compiler_params=_sc_params,
    name="sc_edge_scatter",
)
def _scat_kernel(h2t_hbm, pk_hbm, zeros_hbm, out_hbm, hv, accv, pkv):
    c = lax.axis_index("c")
    s = lax.axis_index("s")
    wid = c * NS + s
    pltpu.sync_copy(h2t_hbm.at[wid], hv)
    pltpu.sync_copy(zeros_hbm, accv)

    def outer(b, carry):
        pltpu.sync_copy(pk_hbm.at[pl.ds(b * BLK, BLK)], pkv)

        def inner(g, carry2):
            pk16 = pkv[pl.ds(g * 16, 16)]
            src = lax.shift_right_logical(pk16, SHIFT)
            dst = jnp.bitwise_and(pk16, MASK)
            for r in range(CP):
                row = jnp.full((16,), r, jnp.int32)
                vals = plsc.load_gather(hv, [row, src])
                plsc.addupdate_scatter(accv, [row, dst], vals)
            return carry2

        lax.fori_loop(0, BLK // 16, inner, 0)
        return carry

    lax.fori_loop(0, NBLK, outer, 0)
    pltpu.sync_copy(accv, out_hbm.at[wid])


def _tc1_body(x_ref, w1_ref, degp_ref, h2_ref, h2t_ref, dinvr_ref,
              dinvc_ref):
    deg = jnp.sum(degp_ref[...], axis=0, keepdims=True)[:, :N] + 1.0
    dinv_row = lax.rsqrt(deg)
    dinv_col = dinv_row.T
    h = jnp.dot(x_ref[...], w1_ref[...], preferred_element_type=jnp.float32)
    h2 = h * dinv_col
    h2_ref[...] = h2
    h2t_ref[...] = h2.T.reshape(NW, CP, N)
    dinvr_ref[...] = dinv_row
    dinvc_ref[...] = dinv_col


def _tc2_body(h2_ref, accp_ref, dinvc_ref, w2_ref, b1_ref,
              h2b_ref, h2bt_ref):
    dinv = dinvc_ref[...]
    acc_t = accp_ref[...].reshape(D, ROWS)[:, :N]
    acc = acc_t.T + h2_ref[...]
    y1 = jnp.maximum(acc * dinv + b1_ref[...], 0.0)
    h = jnp.dot(y1, w2_ref[...], preferred_element_type=jnp.float32)
    h2b = h * dinv
    h2b_ref[...] = h2b
    h2bt_ref[...] = h2b.T.reshape(NW, CP, N)


def _tc3_body(h2b_ref, accp_ref, dinvc_ref, b2_ref, lnw_ref, lnb_ref,
              out_ref):
    acc_t = accp_ref[...].reshape(D, ROWS)[:, :N]
    acc = acc_t.T + h2b_ref[...]
    y = jnp.maximum(acc * dinvc_ref[...] + b2_ref[...], 0.0)
    m = jnp.mean(y)
    o = y - m
    v = jnp.mean(o * o)
    out_ref[...] = (o / jnp.sqrt(v + 1e-5)) * lnw_ref[...] + lnb_ref[...]


_tc1 = pl.pallas_call(
    _tc1_body,
    out_shape=(jax.ShapeDtypeStruct((N, D), jnp.float32),
               jax.ShapeDtypeStruct((NW, CP, N), jnp.float32),
               jax.ShapeDtypeStruct((1, N), jnp.float32),
               jax.ShapeDtypeStruct((N, 1), jnp.float32)),
    name="tc_h2_dinv",
)

_tc2 = pl.pallas_call(
    _tc2_body,
    out_shape=(jax.ShapeDtypeStruct((N, D), jnp.float32),
               jax.ShapeDtypeStruct((NW, CP, N), jnp.float32)),
    name="tc_layer1",
)

_tc3 = pl.pallas_call(
    _tc3_body,
    out_shape=jax.ShapeDtypeStruct((N, D), jnp.float32),
    name="tc_layer2_ln",
)


def kernel(node_features, edges, W1, b1, W2, b2, ln_w, ln_b):
    src = edges[0]
    dst = edges[1]
    pk = src * (1 << SHIFT) + dst
    pad = EPAD - E
    pkp = jnp.concatenate([pk, jnp.full((pad,), N, jnp.int32)])
    zeros_deg = jnp.zeros((ROWS,), jnp.float32)
    zeros_acc = jnp.zeros((CP, ROWS), jnp.float32)

    degp = _deg_kernel(pkp, zeros_deg)
    h2, h2t, dinvr, dinvc = _tc1(node_features, W1, degp)
    del dinvr
    acc1 = _scat_kernel(h2t, pkp, zeros_acc)
    h2b, h2bt = _tc2(h2, acc1, dinvc, W2, b1.reshape(1, D))
    acc2 = _scat_kernel(h2bt, pkp, zeros_acc)
    out = _tc3(h2b, acc2, dinvc, b2.reshape(1, D),
               ln_w.reshape(1, 1), ln_b.reshape(1, 1))
    return out


# speedup vs baseline: 7.4924x; 1.0080x over previous
import functools

import jax
import jax.numpy as jnp
from jax import lax
from jax.experimental import pallas as pl
from jax.experimental.pallas import tpu as pltpu
from jax.experimental.pallas import tpu_sc as plsc

N = 10000
E = 320000
D = 128
NC = 2
NS = 16
NW = NC * NS
CP = D // NW
BLK = 4096
NBLK = -(-E // BLK)
EPAD = NBLK * BLK
EPT = EPAD // NW
ROWS = 10016
SHIFT = 14
MASK = (1 << SHIFT) - 1

_mesh = plsc.VectorSubcoreMesh(core_axis_name="c", subcore_axis_name="s")
_sc_params = pltpu.CompilerParams(use_tc_tiling_on_sc=False,
                                  needs_layout_passes=False)


@functools.partial(
    pl.kernel,
    out_type=jax.ShapeDtypeStruct((NW, ROWS), jnp.float32),
    mesh=_mesh,
    scratch_types=[
        pltpu.VMEM((EPT,), jnp.int32),
        pltpu.VMEM((ROWS,), jnp.float32),
    ],
    compiler_params=_sc_params,
    name="sc_degree",
)
def _deg_kernel(pk_hbm, zeros_hbm, out_hbm, pkv, degv):
    c = lax.axis_index("c")
    s = lax.axis_index("s")
    wid = c * NS + s
    pltpu.sync_copy(pk_hbm.at[pl.ds(wid * EPT, EPT)], pkv)
    pltpu.sync_copy(zeros_hbm, degv)
    ones = jnp.ones((16,), jnp.float32)

    def go(g, carry):
        pk16 = pkv[pl.ds(g * 16, 16)]
        dst = jnp.bitwise_and(pk16, MASK)
        plsc.addupdate_scatter(degv, [dst], ones)
        return carry

    lax.fori_loop(0, EPT // 16, go, 0)
    pltpu.sync_copy(degv, out_hbm.at[wid])


UNROLL = 8


@functools.partial(
    pl.kernel,
    out_type=jax.ShapeDtypeStruct((D, ROWS), jnp.float32),
    mesh=_mesh,
    scratch_types=[
        [pltpu.VMEM((N,), jnp.float32) for _ in range(CP)],
        [pltpu.VMEM((ROWS,), jnp.float32) for _ in range(CP)],
        pltpu.VMEM((BLK,), jnp.int32),
    ],
    compiler_params=_sc_params,
    name="sc_edge_scatter",
)
def _scat_kernel(h2t_hbm, pk_hbm, zeros_hbm, out_hbm, hv, accv, pkv):
    c = lax.axis_index("c")
    s = lax.axis_index("s")
    wid = c * NS + s
    for r in range(CP):
        pltpu.sync_copy(h2t_hbm.at[wid * CP + r], hv[r])
        pltpu.sync_copy(zeros_hbm, accv[r])

    def outer(b, carry):
        pltpu.sync_copy(pk_hbm.at[pl.ds(b * BLK, BLK)], pkv)

        def inner(g, carry2):
            base = g * (16 * UNROLL)
            for u in range(UNROLL):
                pk16 = pkv[pl.ds(base + u * 16, 16)]
                src = lax.shift_right_logical(pk16, SHIFT)
                dst = jnp.bitwise_and(pk16, MASK)
                for r in range(CP):
                    vals = plsc.load_gather(hv[r], [src])
                    plsc.addupdate_scatter(accv[r], [dst], vals)
            return carry2

        lax.fori_loop(0, BLK // (16 * UNROLL), inner, 0)
        return carry

    lax.fori_loop(0, NBLK, outer, 0)
    for r in range(CP):
        pltpu.sync_copy(accv[r], out_hbm.at[wid * CP + r])


def _tc1_body(x_ref, w1_ref, degp_ref, h2_ref, h2t_ref, dinvr_ref,
              dinvc_ref):
    deg = jnp.sum(degp_ref[...], axis=0, keepdims=True)[:, :N] + 1.0
    dinv_row = lax.rsqrt(deg)
    dinv_col = dinv_row.T
    h = jnp.dot(x_ref[...], w1_ref[...], preferred_element_type=jnp.float32)
    h2 = h * dinv_col
    h2_ref[...] = h2
    h2t_ref[...] = h2.T
    dinvr_ref[...] = dinv_row
    dinvc_ref[...] = dinv_col


def _tc2_body(h2_ref, accp_ref, dinvc_ref, w2_ref, b1_ref,
              h2b_ref, h2bt_ref):
    dinv = dinvc_ref[...]
    acc_t = accp_ref[:, :N]
    acc = acc_t.T + h2_ref[...]
    y1 = jnp.maximum(acc * dinv + b1_ref[...], 0.0)
    h = jnp.dot(y1, w2_ref[...], preferred_element_type=jnp.float32)
    h2b = h * dinv
    h2b_ref[...] = h2b
    h2bt_ref[...] = h2b.T


def _tc3_body(h2b_ref, accp_ref, dinvc_ref, b2_ref, lnw_ref, lnb_ref,
              out_ref):
    acc_t = accp_ref[:, :N]
    acc = acc_t.T + h2b_ref[...]
    y = jnp.maximum(acc * dinvc_ref[...] + b2_ref[...], 0.0)
    m = jnp.mean(y)
    o = y - m
    v = jnp.mean(o * o)
    out_ref[...] = (o / jnp.sqrt(v + 1e-5)) * lnw_ref[...] + lnb_ref[...]


_tc1 = pl.pallas_call(
    _tc1_body,
    out_shape=(jax.ShapeDtypeStruct((N, D), jnp.float32),
               jax.ShapeDtypeStruct((D, N), jnp.float32),
               jax.ShapeDtypeStruct((1, N), jnp.float32),
               jax.ShapeDtypeStruct((N, 1), jnp.float32)),
    name="tc_h2_dinv",
)

_tc2 = pl.pallas_call(
    _tc2_body,
    out_shape=(jax.ShapeDtypeStruct((N, D), jnp.float32),
               jax.ShapeDtypeStruct((D, N), jnp.float32)),
    name="tc_layer1",
)

_tc3 = pl.pallas_call(
    _tc3_body,
    out_shape=jax.ShapeDtypeStruct((N, D), jnp.float32),
    name="tc_layer2_ln",
)


def kernel(node_features, edges, W1, b1, W2, b2, ln_w, ln_b):
    src = edges[0]
    dst = edges[1]
    pk = src * (1 << SHIFT) + dst
    pad = EPAD - E
    pkp = jnp.concatenate([pk, jnp.full((pad,), N, jnp.int32)])
    zeros_row = jnp.zeros((ROWS,), jnp.float32)

    degp = _deg_kernel(pkp, zeros_row)
    h2, h2t, dinvr, dinvc = _tc1(node_features, W1, degp)
    del dinvr
    acc1 = _scat_kernel(h2t, pkp, zeros_row)
    h2b, h2bt = _tc2(h2, acc1, dinvc, W2, b1.reshape(1, D))
    acc2 = _scat_kernel(h2bt, pkp, zeros_row)
    out = _tc3(h2b, acc2, dinvc, b2.reshape(1, D),
               ln_w.reshape(1, 1), ln_b.reshape(1, 1))
    return out
